# baseline (speedup 1.0000x reference)
import functools

import jax
import jax.numpy as jnp
from jax.experimental import pallas as pl
from jax.experimental.pallas import tpu as pltpu
from jax.experimental.pallas import tpu_sc as plsc

V_SIZE = 100000
N_POS = 50
EMB = 128
HID = 256

_GATHER_WINDOW = 128


def _sc_gather(table, idx_rows):
    num_windows = idx_rows.shape[0]
    num_indices = num_windows * _GATHER_WINDOW
    mesh = plsc.VectorSubcoreMesh(core_axis_name="c", subcore_axis_name="s")

    @functools.partial(
        pl.kernel,
        out_type=jax.ShapeDtypeStruct((num_indices, EMB), table.dtype),
        mesh=mesh,
    )
    def gather_kernel(table_hbm, idx_hbm, out_hbm):
        def body(idx_vmem, out_vmem):
            pltpu.sync_copy(table_hbm.at[idx_vmem.at[0]], out_vmem)

        pltpu.emit_pipeline(
            body,
            grid=(num_windows,),
            in_specs=[
                pl.BlockSpec((1, _GATHER_WINDOW), index_map=lambda i: (i, 0))
            ],
            out_specs=[
                pl.BlockSpec((_GATHER_WINDOW, EMB), index_map=lambda i: (i, 0))
            ],
            core_axis_name=("c", "s"),
            dimension_semantics=(pltpu.PARALLEL,),
        )(idx_hbm, out_hbm)

    return gather_kernel(table, idx_rows)


def _mlp_body_first(x_ref, p_ref, w1_ref, b1_ref, w2_ref, b2_ref, o_ref):
    h = x_ref[...] + p_ref[...]
    a = jnp.dot(h, w1_ref[...], preferred_element_type=jnp.float32)
    a = jnp.maximum(a + b1_ref[...], 0.0)
    o = jnp.dot(a, w2_ref[...], preferred_element_type=jnp.float32)
    o = o + b2_ref[...]
    o_ref[...] = o.reshape(o_ref.shape)


def _mlp_body_alias(buf_ref, x_ref, p_ref, w1_ref, b1_ref, w2_ref, b2_ref,
                    o_ref):
    del buf_ref
    _mlp_body_first(x_ref, p_ref, w1_ref, b1_ref, w2_ref, b2_ref, o_ref)


def _mlp_chunk(buf, chunk_id, gathered, pos_tiled, W1, b1, W2, b2,
               block_rows, batch):
    n_rows = gathered.shape[0]
    bb = block_rows // N_POS
    blocks_per_chunk = n_rows // block_rows
    block0 = chunk_id * blocks_per_chunk

    data_specs = [
        pl.BlockSpec((block_rows, EMB), lambda i: (i, 0)),
        pl.BlockSpec((block_rows, EMB), lambda i: (0, 0)),
        pl.BlockSpec((EMB, HID), lambda i: (0, 0)),
        pl.BlockSpec((1, HID), lambda i: (0, 0)),
        pl.BlockSpec((HID, EMB), lambda i: (0, 0)),
        pl.BlockSpec((1, EMB), lambda i: (0, 0)),
    ]
    out_spec = pl.BlockSpec(
        (bb, N_POS, EMB), lambda i: (block0 + i, 0, 0)
    )
    out_type = jax.ShapeDtypeStruct((batch, N_POS, EMB), jnp.float32)
    cp = pltpu.CompilerParams(dimension_semantics=("parallel",))
    args = (gathered, pos_tiled, W1, b1, W2, b2)
    if buf is None:
        return pl.pallas_call(
            _mlp_body_first,
            grid=(blocks_per_chunk,),
            in_specs=data_specs,
            out_specs=out_spec,
            out_shape=out_type,
            compiler_params=cp,
        )(*args)
    return pl.pallas_call(
        _mlp_body_alias,
        grid=(blocks_per_chunk,),
        in_specs=[pl.BlockSpec(memory_space=pl.ANY)] + data_specs,
        out_specs=out_spec,
        out_shape=out_type,
        input_output_aliases={0: 0},
        compiler_params=cp,
    )(buf, *args)


_N_CHUNKS = 4


def kernel(x, mask, emb_table, pos_table, W1, b1, W2, b2):
    B, N = x.shape
    xi = jnp.where(mask, V_SIZE - 1, x).astype(jnp.int32)
    idx_rows = xi.reshape(-1, _GATHER_WINDOW)
    block_rows = 12800
    pos_tiled = jnp.tile(pos_table, (block_rows // N, 1))
    b1r = b1.reshape(1, HID)
    b2r = b2.reshape(1, EMB)

    windows_per_chunk = idx_rows.shape[0] // _N_CHUNKS
    buf = None
    for k in range(_N_CHUNKS):
        idx_k = jax.lax.slice_in_dim(
            idx_rows, k * windows_per_chunk, (k + 1) * windows_per_chunk
        )
        gathered = _sc_gather(emb_table, idx_k)
        buf = _mlp_chunk(
            buf, k, gathered, pos_tiled, W1, b1r, W2, b2r, block_rows, B
        )
    return buf


# speedup vs baseline: 1.0073x; 1.0073x over previous
import functools

import jax
import jax.numpy as jnp
from jax.experimental import pallas as pl
from jax.experimental.pallas import tpu as pltpu
from jax.experimental.pallas import tpu_sc as plsc

V_SIZE = 100000
N_POS = 50
EMB = 128
HID = 256

_GATHER_WINDOW = 128


def _sc_gather(table, idx_rows):
    num_windows = idx_rows.shape[0]
    num_indices = num_windows * _GATHER_WINDOW
    mesh = plsc.VectorSubcoreMesh(core_axis_name="c", subcore_axis_name="s")

    @functools.partial(
        pl.kernel,
        out_type=jax.ShapeDtypeStruct((num_indices, EMB), table.dtype),
        mesh=mesh,
    )
    def gather_kernel(table_hbm, idx_hbm, out_hbm):
        def body(idx_vmem, out_vmem):
            pltpu.sync_copy(table_hbm.at[idx_vmem.at[0]], out_vmem)

        pltpu.emit_pipeline(
            body,
            grid=(num_windows,),
            in_specs=[
                pl.BlockSpec((1, _GATHER_WINDOW), index_map=lambda i: (i, 0))
            ],
            out_specs=[
                pl.BlockSpec((_GATHER_WINDOW, EMB), index_map=lambda i: (i, 0))
            ],
            core_axis_name=("c", "s"),
            dimension_semantics=(pltpu.PARALLEL,),
        )(idx_hbm, out_hbm)

    return gather_kernel(table, idx_rows)


def _mlp_body_first(x_ref, p_ref, w1_ref, b1_ref, w2_ref, b2_ref, o_ref):
    h = x_ref[...] + p_ref[...]
    a = jnp.dot(h, w1_ref[...], preferred_element_type=jnp.float32)
    a = jnp.maximum(a + b1_ref[...], 0.0)
    o = jnp.dot(a, w2_ref[...], preferred_element_type=jnp.float32)
    o = o + b2_ref[...]
    o_ref[...] = o.reshape(o_ref.shape)


def _mlp_body_alias(buf_ref, x_ref, p_ref, w1_ref, b1_ref, w2_ref, b2_ref,
                    o_ref):
    del buf_ref
    _mlp_body_first(x_ref, p_ref, w1_ref, b1_ref, w2_ref, b2_ref, o_ref)


def _mlp_chunk(buf, block0, gathered, pos_tiled, W1, b1, W2, b2,
               block_rows, batch):
    n_rows = gathered.shape[0]
    bb = block_rows // N_POS
    blocks_per_chunk = n_rows // block_rows

    data_specs = [
        pl.BlockSpec((block_rows, EMB), lambda i: (i, 0)),
        pl.BlockSpec((block_rows, EMB), lambda i: (0, 0)),
        pl.BlockSpec((EMB, HID), lambda i: (0, 0)),
        pl.BlockSpec((1, HID), lambda i: (0, 0)),
        pl.BlockSpec((HID, EMB), lambda i: (0, 0)),
        pl.BlockSpec((1, EMB), lambda i: (0, 0)),
    ]
    out_spec = pl.BlockSpec(
        (bb, N_POS, EMB), lambda i: (block0 + i, 0, 0)
    )
    out_type = jax.ShapeDtypeStruct((batch, N_POS, EMB), jnp.float32)
    cp = pltpu.CompilerParams(dimension_semantics=("parallel",))
    args = (gathered, pos_tiled, W1, b1, W2, b2)
    if buf is None:
        return pl.pallas_call(
            _mlp_body_first,
            grid=(blocks_per_chunk,),
            in_specs=data_specs,
            out_specs=out_spec,
            out_shape=out_type,
            compiler_params=cp,
        )(*args)
    return pl.pallas_call(
        _mlp_body_alias,
        grid=(blocks_per_chunk,),
        in_specs=[pl.BlockSpec(memory_space=pl.ANY)] + data_specs,
        out_specs=out_spec,
        out_shape=out_type,
        input_output_aliases={0: 0},
        compiler_params=cp,
    )(buf, *args)


_CHUNK_BATCHES = (1280, 1280, 1024, 512)


def kernel(x, mask, emb_table, pos_table, W1, b1, W2, b2):
    B, N = x.shape
    xi = jnp.where(mask, V_SIZE - 1, x).astype(jnp.int32)
    idx_rows = xi.reshape(-1, _GATHER_WINDOW)
    block_rows = 6400
    bb = block_rows // N
    pos_tiled = jnp.tile(pos_table, (block_rows // N, 1))
    b1r = b1.reshape(1, HID)
    b2r = b2.reshape(1, EMB)

    buf = None
    batch0 = 0
    for chunk_batch in _CHUNK_BATCHES:
        idx_k = jax.lax.slice_in_dim(
            idx_rows, batch0 * N // _GATHER_WINDOW,
            (batch0 + chunk_batch) * N // _GATHER_WINDOW,
        )
        gathered = _sc_gather(emb_table, idx_k)
        buf = _mlp_chunk(
            buf, batch0 // bb, gathered, pos_tiled, W1, b1r, W2, b2r,
            block_rows, B,
        )
        batch0 += chunk_batch
    return buf


# speedup vs baseline: 1.0144x; 1.0070x over previous
import functools

import jax
import jax.numpy as jnp
from jax.experimental import pallas as pl
from jax.experimental.pallas import tpu as pltpu
from jax.experimental.pallas import tpu_sc as plsc

V_SIZE = 100000
N_POS = 50
EMB = 128
HID = 256

_GATHER_WINDOW = 128


def _sc_gather(table, idx_rows):
    num_windows = idx_rows.shape[0]
    num_indices = num_windows * _GATHER_WINDOW
    mesh = plsc.VectorSubcoreMesh(core_axis_name="c", subcore_axis_name="s")

    @functools.partial(
        pl.kernel,
        out_type=jax.ShapeDtypeStruct((num_indices, EMB), table.dtype),
        mesh=mesh,
    )
    def gather_kernel(table_hbm, idx_hbm, out_hbm):
        def body(idx_vmem, out_vmem):
            pltpu.sync_copy(table_hbm.at[idx_vmem.at[0]], out_vmem)

        pltpu.emit_pipeline(
            body,
            grid=(num_windows,),
            in_specs=[
                pl.BlockSpec((1, _GATHER_WINDOW), index_map=lambda i: (i, 0))
            ],
            out_specs=[
                pl.BlockSpec((_GATHER_WINDOW, EMB), index_map=lambda i: (i, 0))
            ],
            core_axis_name=("c", "s"),
            dimension_semantics=(pltpu.PARALLEL,),
        )(idx_hbm, out_hbm)

    return gather_kernel(table, idx_rows)


def _mlp_body_first(x_ref, p_ref, w1_ref, b1_ref, w2_ref, b2_ref, o_ref):
    h = x_ref[...] + p_ref[...]
    a = jnp.dot(h, w1_ref[...], preferred_element_type=jnp.float32)
    a = jnp.maximum(a + b1_ref[...], 0.0)
    o = jnp.dot(a, w2_ref[...], preferred_element_type=jnp.float32)
    o = o + b2_ref[...]
    o_ref[...] = o.reshape(o_ref.shape)


def _mlp_body_alias(buf_ref, x_ref, p_ref, w1_ref, b1_ref, w2_ref, b2_ref,
                    o_ref):
    del buf_ref
    _mlp_body_first(x_ref, p_ref, w1_ref, b1_ref, w2_ref, b2_ref, o_ref)


def _mlp_chunk(buf, block0, gathered, pos_tiled, W1, b1, W2, b2,
               block_rows, batch):
    n_rows = gathered.shape[0]
    bb = block_rows // N_POS
    blocks_per_chunk = n_rows // block_rows

    data_specs = [
        pl.BlockSpec((block_rows, EMB), lambda i: (i, 0)),
        pl.BlockSpec((block_rows, EMB), lambda i: (0, 0)),
        pl.BlockSpec((EMB, HID), lambda i: (0, 0)),
        pl.BlockSpec((1, HID), lambda i: (0, 0)),
        pl.BlockSpec((HID, EMB), lambda i: (0, 0)),
        pl.BlockSpec((1, EMB), lambda i: (0, 0)),
    ]
    out_spec = pl.BlockSpec(
        (bb, N_POS, EMB), lambda i: (block0 + i, 0, 0)
    )
    out_type = jax.ShapeDtypeStruct((batch, N_POS, EMB), jnp.float32)
    cp = pltpu.CompilerParams(dimension_semantics=("parallel",))
    args = (gathered, pos_tiled, W1, b1, W2, b2)
    if buf is None:
        return pl.pallas_call(
            _mlp_body_first,
            grid=(blocks_per_chunk,),
            in_specs=data_specs,
            out_specs=out_spec,
            out_shape=out_type,
            compiler_params=cp,
        )(*args)
    return pl.pallas_call(
        _mlp_body_alias,
        grid=(blocks_per_chunk,),
        in_specs=[pl.BlockSpec(memory_space=pl.ANY)] + data_specs,
        out_specs=out_spec,
        out_shape=out_type,
        input_output_aliases={0: 0},
        compiler_params=cp,
    )(buf, *args)


_CHUNK_BATCHES = (1536, 1536, 1024)


def kernel(x, mask, emb_table, pos_table, W1, b1, W2, b2):
    B, N = x.shape
    xi = jnp.where(mask, V_SIZE - 1, x).astype(jnp.int32)
    idx_rows = xi.reshape(-1, _GATHER_WINDOW)
    block_rows = 6400
    bb = block_rows // N
    pos_tiled = jnp.tile(pos_table, (block_rows // N, 1))
    b1r = b1.reshape(1, HID)
    b2r = b2.reshape(1, EMB)

    buf = None
    batch0 = 0
    for chunk_batch in _CHUNK_BATCHES:
        idx_k = jax.lax.slice_in_dim(
            idx_rows, batch0 * N // _GATHER_WINDOW,
            (batch0 + chunk_batch) * N // _GATHER_WINDOW,
        )
        gathered = _sc_gather(emb_table, idx_k)
        buf = _mlp_chunk(
            buf, batch0 // bb, gathered, pos_tiled, W1, b1r, W2, b2r,
            block_rows, B,
        )
        batch0 += chunk_batch
    return buf
